# baseline (speedup 1.0000x reference)
import functools

import jax
import jax.numpy as jnp
from jax import lax
from jax.experimental import pallas as pl
from jax.experimental.pallas import tpu as pltpu
from jax.experimental.pallas import tpu_sc as plsc

N_P = 16
PATCH = 96
HS = 128
WS = 128
HH = 1024
WH = 1024
CH = 3
NF = HS * WS
LANE = 16

TROW = 128
TPIMROW = WH * CH // TROW
WTAB = 4
PCHW = CH * PATCH * PATCH
SLAB_ROWS = PATCH * WTAB
ZROW = SLAB_ROWS


def _topk_kernel(att_ref, gum_ref, idx_ref, sa_ref, v_ref):
    att = att_ref[...]
    v_ref[...] = att + gum_ref[...]
    iota = jax.lax.broadcasted_iota(jnp.int32, att.shape, 1)
    for k in range(N_P):
        v = v_ref[...]
        m = jnp.max(v, axis=1, keepdims=True)
        idx = jnp.min(jnp.where(v == m, iota, NF), axis=1, keepdims=True)
        hit = iota == idx
        sa = jnp.sum(jnp.where(hit, att, 0.0), axis=1, keepdims=True)
        idx_ref[:, k : k + 1] = idx
        sa_ref[:, k : k + 1] = sa
        v_ref[...] = jnp.where(hit, -jnp.inf, v)


def _sc_patch_body(nc, pw, xh2, sflat, out_hbm, sidx_v, slab_v, out_v, sem):
    wid = lax.axis_index("s") * nc + lax.axis_index("c")
    lane = lax.iota(jnp.int32, LANE)

    zeros = jnp.zeros((LANE,), jnp.float32)
    for z in range(TROW // LANE):
        slab_v[ZROW, pl.ds(z * LANE, LANE)] = zeros

    base = pl.multiple_of((wid * pw // LANE) * LANE, 8)
    pltpu.sync_copy(sflat.at[pl.ds(base, LANE)], sidx_v)
    q = wid * pw - base

    for k in range(pw):
        pflat = wid * pw + k
        b = pflat // N_P
        sv = plsc.load_gather(sidx_v, [jnp.broadcast_to(q + k, (LANE,))])
        sxv = sv // WS
        syv = sv - sxv * WS
        r0v = sxv * 8 - 44
        c0v = syv * 8 - 44
        f0av = jnp.minimum(
            (3 * jnp.clip(c0v, 0, 928)) // TROW * TROW, WH * CH - WTAB * TROW
        )

        tcolv = f0av // TROW
        lrow = lane >> 2
        ltab = lane & (WTAB - 1)

        def fire(i4, c):
            rsrcv = jnp.clip(r0v + i4 * 4 + lrow, 0, HH - 1)
            idxv = (b * HH + rsrcv) * TPIMROW + tcolv + ltab
            pltpu.async_copy(
                xh2.at[idxv],
                slab_v.at[pl.ds(pl.multiple_of(i4 * LANE, 8), LANE)],
                sem,
            )
            return c

        lax.fori_loop(0, SLAB_ROWS // LANE, fire, 0)
        pltpu.make_async_copy(
            xh2.at[pl.ds(0, SLAB_ROWS)], slab_v.at[pl.ds(0, SLAB_ROWS)], sem
        ).wait()

        o0s = []
        cvs = []
        for jb in range(PATCH // LANE):
            col = c0v + jb * LANE + lane
            cvs.append((col >= 0) & (col < WH))
            o0s.append(jnp.clip(3 * col - f0av, 0, WTAB * TROW - 3))

        def row(i, c):
            riv = r0v + i
            rvv = (riv >= 0) & (riv < HH)
            for jb in range(PATCH // LANE):
                valid = cvs[jb] & rvv
                for ch in range(CH):
                    o = o0s[jb] + ch
                    idxr = jnp.where(valid, (o >> 7) + i * WTAB, ZROW)
                    idxc = o & (TROW - 1)
                    val = plsc.load_gather(slab_v, [idxr, idxc])
                    off = ch * PATCH * PATCH + i * PATCH + jb * LANE
                    out_v[pl.ds(pl.multiple_of(off, 8), LANE)] = val
            return c

        lax.fori_loop(0, PATCH, row, 0)
        pltpu.sync_copy(
            out_v, out_hbm.at[pl.ds(pl.multiple_of(pflat * PCHW, 8), PCHW)]
        )


def kernel(x_low, x_high, attention):
    B = attention.shape[0]
    att2 = attention.reshape(B, NF)
    u = jax.random.uniform(jax.random.key(42), (B, NF), minval=1e-8, maxval=1.0)
    gum = -jnp.log(-jnp.log(u))

    sflat, sampled_att = pl.pallas_call(
        _topk_kernel,
        out_shape=[
            jax.ShapeDtypeStruct((B, N_P), jnp.int32),
            jax.ShapeDtypeStruct((B, N_P), jnp.float32),
        ],
        scratch_shapes=[pltpu.VMEM((B, NF), jnp.float32)],
    )(att2, gum)

    mesh = plsc.VectorSubcoreMesh(core_axis_name="c", subcore_axis_name="s")
    nw = mesh.num_cores * mesh.num_subcores
    pw = (B * N_P) // nw
    xh2 = x_high.reshape(-1, TROW)

    sc_patches = pl.kernel(
        functools.partial(_sc_patch_body, mesh.num_cores, pw),
        out_type=jax.ShapeDtypeStruct((B * N_P * PCHW,), jnp.float32),
        mesh=mesh,
        scratch_types=[
            pltpu.VMEM((LANE,), jnp.int32),
            pltpu.VMEM((SLAB_ROWS + 1, TROW), jnp.float32),
            pltpu.VMEM((PCHW,), jnp.float32),
            pltpu.SemaphoreType.DMA,
        ],
        compiler_params=pltpu.CompilerParams(needs_layout_passes=False),
    )
    patches = sc_patches(xh2, sflat.reshape(-1))
    return patches.reshape(B, N_P, CH, PATCH, PATCH), sampled_att
